# baseline (speedup 1.0000x reference)
import functools

import jax
import jax.numpy as jnp
from jax import lax
from jax.experimental import pallas as pl
from jax.experimental.pallas import tpu as pltpu
from jax.experimental.pallas import tpu_sc as plsc

B = 16384
D = 64
H = 4 * D
NC, NS = 2, 16
NW = NC * NS
BPW = B // NW
VA = 1000000
CHB = 512
S = 977 * CHB
_mesh = plsc.VectorSubcoreMesh(
    core_axis_name="c", subcore_axis_name="s", num_cores=NC, num_subcores=NS
)


def _tp_body(l_ref, r_ref, o_ref):
    o_ref[...] = jnp.concatenate(
        [l_ref[...].T, r_ref[...].T], axis=1)


_tp = pl.pallas_call(
    _tp_body,
    grid=(S // CHB,),
    in_specs=[
        pl.BlockSpec((D, CHB), lambda i: (0, i)),
        pl.BlockSpec((D, CHB), lambda i: (0, i + S // CHB)),
    ],
    out_specs=pl.BlockSpec((CHB, 2 * D), lambda i: (i, 0)),
    out_shape=jax.ShapeDtypeStruct((S, 2 * D), jnp.float32),
    compiler_params=pltpu.CompilerParams(
        dimension_semantics=("parallel",),
    ),
)


def _fire_rows(tbl, idx_v, rows_v, sem, col0, width):
    def body(g, _):
        i0 = g * 16
        vl = idx_v[pl.ds(i0, 16)]
        for j in range(16):
            pltpu.async_copy(
                tbl.at[vl[j]], rows_v.at[i0 + j, pl.ds(col0, width)], sem)
        return ()

    lax.fori_loop(0, BPW // 16, body, ())


@functools.partial(
    pl.kernel,
    out_type=(
        jax.ShapeDtypeStruct((B, D), jnp.float32),
        jax.ShapeDtypeStruct((B, D), jnp.float32),
        jax.ShapeDtypeStruct((B, D), jnp.float32),
    ),
    mesh=_mesh,
    scratch_types=[
        pltpu.VMEM((BPW,), jnp.int32),
        pltpu.VMEM((BPW, D), jnp.float32),
        pltpu.SemaphoreType.DMA,
    ],
)
def _sc_gather_small(tg, tal, tc_, ig, ial, ic, og, oal, oc,
                     idx_v, rows_v, sem):
    wid = lax.axis_index("s") * NC + lax.axis_index("c")
    base = wid * BPW
    for tbl, idx_hbm, out_hbm in ((tg, ig, og), (tal, ial, oal),
                                  (tc_, ic, oc)):
        pltpu.sync_copy(idx_hbm.at[pl.ds(base, BPW)], idx_v)
        _fire_rows(tbl, idx_v, rows_v, sem, 0, D)
        pltpu.make_async_copy(out_hbm.at[pl.ds(base, BPW)], rows_v,
                              sem).wait()
        pltpu.sync_copy(rows_v, out_hbm.at[pl.ds(base, BPW)])


@functools.partial(
    pl.kernel,
    out_type=jax.ShapeDtypeStruct((B, 2 * D), jnp.float32),
    mesh=_mesh,
    scratch_types=[
        pltpu.VMEM((BPW,), jnp.int32),
        pltpu.VMEM((BPW, 2 * D), jnp.float32),
        pltpu.SemaphoreType.DMA,
    ],
)
def _sc_gather_artist(taP, ia2, oa, idx_v, rows_v, sem):
    wid = lax.axis_index("s") * NC + lax.axis_index("c")
    base = wid * BPW
    pltpu.sync_copy(ia2.at[pl.ds(base, BPW)], idx_v)
    _fire_rows(taP, idx_v, rows_v, sem, 0, 2 * D)
    pltpu.make_async_copy(oa.at[pl.ds(base, BPW)], rows_v, sem).wait()
    pltpu.sync_copy(rows_v, oa.at[pl.ds(base, BPW)])


TB = 2048


def _mm_body(xg, xa, xal, xc, sa, w_ref, b_ref, o_ref):
    xa_v = xa[...]
    s = sa[...]
    e_a = xa_v[:, :D] * (1.0 - s) + xa_v[:, D:] * s
    x = jnp.concatenate([xg[...], e_a, xal[...], xc[...]], axis=1)
    acc = lax.dot_general(x, w_ref[...], (((1,), (1,)), ((), ())),
                          preferred_element_type=jnp.float32)
    o_ref[...] = acc + b_ref[...]


_mm = pl.pallas_call(
    _mm_body,
    grid=(B // TB,),
    in_specs=[
        pl.BlockSpec((TB, D), lambda i: (i, 0)),
        pl.BlockSpec((TB, 2 * D), lambda i: (i, 0)),
        pl.BlockSpec((TB, D), lambda i: (i, 0)),
        pl.BlockSpec((TB, D), lambda i: (i, 0)),
        pl.BlockSpec((TB, 1), lambda i: (i, 0)),
        pl.BlockSpec((H, H), lambda i: (0, 0)),
        pl.BlockSpec((1, H), lambda i: (0, 0)),
    ],
    out_specs=pl.BlockSpec((TB, H), lambda i: (i, 0)),
    out_shape=jax.ShapeDtypeStruct((B, H), jnp.float32),
    compiler_params=pltpu.CompilerParams(
        dimension_semantics=("parallel",),
    ),
)


def kernel(emb_genre, emb_artist, emb_album, emb_country, W, b,
           idx_genre, idx_artist, idx_album, idx_country):
    taT = emb_artist.T
    taP = _tp(taT, taT)
    ia = idx_artist.astype(jnp.int32)
    hi = ia >= S
    ia2 = jnp.where(hi, ia - S, ia)
    sa = hi.astype(jnp.float32).reshape(B, 1)

    x_g, x_al, x_c = _sc_gather_small(
        emb_genre, emb_album, emb_country,
        idx_genre.astype(jnp.int32), idx_album.astype(jnp.int32),
        idx_country.astype(jnp.int32),
    )
    x_a = _sc_gather_artist(taP, ia2)
    return _mm(x_g, x_a, x_al, x_c, sa, W, b.reshape(1, H))


# speedup vs baseline: 1.8505x; 1.8505x over previous
import functools

import jax
import jax.numpy as jnp
from jax import lax
from jax.experimental import pallas as pl
from jax.experimental.pallas import tpu as pltpu
from jax.experimental.pallas import tpu_sc as plsc

B = 16384
D = 64
H = 4 * D
NC, NS = 2, 16
NW = NC * NS
BPW = B // NW
VA = 1000000
CHB = 2048
S = 244 * CHB
_mesh = plsc.VectorSubcoreMesh(
    core_axis_name="c", subcore_axis_name="s", num_cores=NC, num_subcores=NS
)


def _tp_body(l_ref, r_ref, o_ref):
    o_ref[...] = jnp.concatenate(
        [l_ref[...].T, r_ref[...].T], axis=1)


_tp = pl.pallas_call(
    _tp_body,
    grid=(S // CHB,),
    in_specs=[
        pl.BlockSpec((D, CHB), lambda i: (0, i)),
        pl.BlockSpec((D, CHB), lambda i: (0, i + S // CHB)),
    ],
    out_specs=pl.BlockSpec((CHB, 2 * D), lambda i: (i, 0)),
    out_shape=jax.ShapeDtypeStruct((S, 2 * D), jnp.float32),
    compiler_params=pltpu.CompilerParams(
        dimension_semantics=("parallel",),
    ),
)


def _fire_rows(tbl, idx_v, rows_v, sem, col0, width):
    def body(g, _):
        i0 = g * 16
        vl = idx_v[pl.ds(i0, 16)]
        for j in range(16):
            pltpu.async_copy(
                tbl.at[vl[j]], rows_v.at[i0 + j, pl.ds(col0, width)], sem)
        return ()

    lax.fori_loop(0, BPW // 16, body, ())


@functools.partial(
    pl.kernel,
    out_type=(
        jax.ShapeDtypeStruct((B, D), jnp.float32),
        jax.ShapeDtypeStruct((B, D), jnp.float32),
        jax.ShapeDtypeStruct((B, D), jnp.float32),
    ),
    mesh=_mesh,
    scratch_types=[
        pltpu.VMEM((BPW,), jnp.int32),
        pltpu.VMEM((BPW, D), jnp.float32),
        pltpu.SemaphoreType.DMA,
    ],
)
def _sc_gather_small(tg, tal, tc_, ig, ial, ic, og, oal, oc,
                     idx_v, rows_v, sem):
    wid = lax.axis_index("s") * NC + lax.axis_index("c")
    base = wid * BPW
    for tbl, idx_hbm, out_hbm in ((tg, ig, og), (tal, ial, oal),
                                  (tc_, ic, oc)):
        pltpu.sync_copy(idx_hbm.at[pl.ds(base, BPW)], idx_v)
        _fire_rows(tbl, idx_v, rows_v, sem, 0, D)
        pltpu.make_async_copy(out_hbm.at[pl.ds(base, BPW)], rows_v,
                              sem).wait()
        pltpu.sync_copy(rows_v, out_hbm.at[pl.ds(base, BPW)])


@functools.partial(
    pl.kernel,
    out_type=jax.ShapeDtypeStruct((B, 2 * D), jnp.float32),
    mesh=_mesh,
    scratch_types=[
        pltpu.VMEM((BPW,), jnp.int32),
        pltpu.VMEM((BPW, 2 * D), jnp.float32),
        pltpu.SemaphoreType.DMA,
    ],
)
def _sc_gather_artist(taP, ia2, oa, idx_v, rows_v, sem):
    wid = lax.axis_index("s") * NC + lax.axis_index("c")
    base = wid * BPW
    pltpu.sync_copy(ia2.at[pl.ds(base, BPW)], idx_v)
    _fire_rows(taP, idx_v, rows_v, sem, 0, 2 * D)
    pltpu.make_async_copy(oa.at[pl.ds(base, BPW)], rows_v, sem).wait()
    pltpu.sync_copy(rows_v, oa.at[pl.ds(base, BPW)])


TB = 2048


def _mm_body(xg, xa, xal, xc, sa, w_ref, b_ref, o_ref):
    xa_v = xa[...]
    s = sa[...]
    e_a = xa_v[:, :D] * (1.0 - s) + xa_v[:, D:] * s
    x = jnp.concatenate([xg[...], e_a, xal[...], xc[...]], axis=1)
    acc = lax.dot_general(x, w_ref[...], (((1,), (1,)), ((), ())),
                          preferred_element_type=jnp.float32)
    o_ref[...] = acc + b_ref[...]


_mm = pl.pallas_call(
    _mm_body,
    grid=(B // TB,),
    in_specs=[
        pl.BlockSpec((TB, D), lambda i: (i, 0)),
        pl.BlockSpec((TB, 2 * D), lambda i: (i, 0)),
        pl.BlockSpec((TB, D), lambda i: (i, 0)),
        pl.BlockSpec((TB, D), lambda i: (i, 0)),
        pl.BlockSpec((TB, 1), lambda i: (i, 0)),
        pl.BlockSpec((H, H), lambda i: (0, 0)),
        pl.BlockSpec((1, H), lambda i: (0, 0)),
    ],
    out_specs=pl.BlockSpec((TB, H), lambda i: (i, 0)),
    out_shape=jax.ShapeDtypeStruct((B, H), jnp.float32),
    compiler_params=pltpu.CompilerParams(
        dimension_semantics=("parallel",),
    ),
)


def kernel(emb_genre, emb_artist, emb_album, emb_country, W, b,
           idx_genre, idx_artist, idx_album, idx_country):
    taT = emb_artist.T
    taP = _tp(taT, taT)
    ia = idx_artist.astype(jnp.int32)
    hi = ia >= S
    ia2 = jnp.where(hi, ia - S, ia)
    sa = hi.astype(jnp.float32).reshape(B, 1)

    x_g, x_al, x_c = _sc_gather_small(
        emb_genre, emb_album, emb_country,
        idx_genre.astype(jnp.int32), idx_album.astype(jnp.int32),
        idx_country.astype(jnp.int32),
    )
    x_a = _sc_gather_artist(taP, ia2)
    return _mm(x_g, x_a, x_al, x_c, sa, W, b.reshape(1, H))
